# Initial kernel scaffold; baseline (speedup 1.0000x reference)
import jax
import jax.numpy as jnp
from jax.experimental import pallas as pl


def kernel(x, edge_index, W1, b1, g1, be1, W2, b2, g2, be2):
    raise NotImplementedError("write your pallas kernel here")


# speedup vs baseline: 2.0749x; 2.0749x over previous
import jax
import jax.numpy as jnp
from jax import lax
from jax.experimental import pallas as pl
from jax.experimental.pallas import tpu as pltpu
from jax.experimental.pallas import tpu_sc as plsc

N = 10000
H = 128
E = 320000
NC = 2
NS = 16
NW = NC * NS
NP = 10240
RPW = NP // NW
C = 128
EP = E + 2 * C

_HALF_BOUNDS = [10080 * k for k in range(1, 12)] + [
    120720, 130560, 140400, 150240]
_SHARD_BOUNDS = _HALF_BOUNDS + [E // 2] + [E // 2 + b for b in _HALF_BOUNDS]


def _sc_agg_body(h_hbm, src_hbm, dst_hbm, bounds_hbm, defer_hbm, out_hbm,
                 acc, srcv, dstv, rows_v, bv, dv, sem):
    c = lax.axis_index("c")
    s = lax.axis_index("s")
    w = c * NS + s
    wbase = w * RPW

    pltpu.sync_copy(bounds_hbm, bv)
    pltpu.sync_copy(defer_hbm, dv)
    b16 = bv[pl.ds(w, 16)]
    lo = b16[0]
    hi = b16[1]
    d16 = dv[pl.ds(4 * w, 16)]
    dlo1 = d16[0]
    dhi1 = d16[1]
    dlo2 = d16[2]
    dhi2 = d16[3]

    def _zrow(r, carry):
        for j in range(H // 16):
            acc[r, pl.ds(j * 16, 16)] = jnp.zeros((16,), jnp.float32)
        return carry

    lax.fori_loop(0, RPW, _zrow, 0)

    start = lo - (lo % 8)
    nchunks = (hi - start + (C - 1)) // C

    def _chunk(i, carry):
        base = pl.multiple_of(start + i * C, 8)
        pltpu.sync_copy(src_hbm.at[pl.ds(base, C)], srcv)
        pltpu.sync_copy(dst_hbm.at[pl.ds(base, C)], dstv.at[pl.ds(0, C)])
        pltpu.async_copy(h_hbm.at[srcv], rows_v, sem).wait()

        def _edge(e, carry2):
            eid = base + e
            valid = (eid >= lo) & (eid < hi)
            valid &= ~((eid >= dlo1) & (eid < dhi1))
            valid &= ~((eid >= dlo2) & (eid < dhi2))
            d = dstv[pl.ds(e, 16)][0]
            dloc = jnp.clip(d - wbase, 0, RPW - 1)
            for j in range(H // 16):
                v16 = rows_v[e, pl.ds(j * 16, 16)]
                v16 = jnp.where(valid, v16, 0.0)
                plsc.addupdate(acc.at[dloc, pl.ds(j * 16, 16)], v16)
            return carry2

        lax.fori_loop(0, C, _edge, 0)
        return carry

    lax.fori_loop(0, nchunks, _chunk, 0)

    for dlo, dhi in ((dlo1, dhi1), (dlo2, dhi2)):
        @pl.when(dhi > dlo)
        def _():
            ab = pl.multiple_of(dlo - (dlo % 8), 8)
            pltpu.sync_copy(src_hbm.at[pl.ds(ab, C)], srcv)
            pltpu.sync_copy(dst_hbm.at[pl.ds(ab, C)],
                            dstv.at[pl.ds(0, C)])
            pltpu.async_copy(h_hbm.at[srcv], rows_v, sem).wait()
            bd = dstv[pl.ds(dlo - ab, 16)][0]
            dloc = jnp.clip(bd - wbase, 0, RPW - 1)

            def _dedge(e, carry2):
                eid = ab + e
                valid = (eid >= dlo) & (eid < dhi)
                out = []
                for j in range(H // 16):
                    v16 = rows_v[e, pl.ds(j * 16, 16)]
                    out.append(carry2[j] + jnp.where(valid, v16, 0.0))
                return tuple(out)

            zero8 = tuple(jnp.zeros((16,), jnp.float32)
                          for _ in range(H // 16))
            tmp = lax.fori_loop(0, C, _dedge, zero8)
            for j in range(H // 16):
                plsc.addupdate(acc.at[dloc, pl.ds(j * 16, 16)], tmp[j])

    pltpu.sync_copy(acc, out_hbm.at[pl.ds(wbase, RPW)])


_sc_agg = pl.kernel(
    _sc_agg_body,
    out_type=jax.ShapeDtypeStruct((NP, H), jnp.float32),
    mesh=plsc.VectorSubcoreMesh(
        core_axis_name="c", subcore_axis_name="s",
        num_cores=NC, num_subcores=NS),
    scratch_types=[
        pltpu.VMEM((RPW, H), jnp.float32),
        pltpu.VMEM((C,), jnp.int32),
        pltpu.VMEM((C + 16,), jnp.int32),
        pltpu.VMEM((C, H), jnp.float32),
        pltpu.VMEM((NW + 16,), jnp.int32),
        pltpu.VMEM((4 * NW + 16,), jnp.int32),
        pltpu.SemaphoreType.DMA,
    ],
)


def _tc_mlp_body(h_ref, agg_ref, w1_ref, b1_ref, g1_ref, be1_ref,
                 w2_ref, b2_ref, g2_ref, be2_ref, out_ref):
    h = h_ref[...] + agg_ref[...]
    z = jnp.dot(h, w1_ref[...], preferred_element_type=jnp.float32)
    z = z + b1_ref[...]
    m = jnp.mean(z, axis=0, keepdims=True)
    d = z - m
    v = jnp.mean(d * d, axis=0, keepdims=True)
    z = d / jnp.sqrt(v + 1e-5) * g1_ref[...] + be1_ref[...]
    z = jnp.maximum(z, 0.0)
    z2 = jnp.dot(z, w2_ref[...], preferred_element_type=jnp.float32)
    z2 = z2 + b2_ref[...]
    m2 = jnp.mean(z2, axis=0, keepdims=True)
    d2 = z2 - m2
    v2 = jnp.mean(d2 * d2, axis=0, keepdims=True)
    out_ref[...] = jnp.maximum(
        d2 / jnp.sqrt(v2 + 1e-5) * g2_ref[...] + be2_ref[...], 0.0)


def _tc_mlp(h, agg, w1, b1, g1, be1, w2, b2, g2, be2):
    return pl.pallas_call(
        _tc_mlp_body,
        out_shape=jax.ShapeDtypeStruct((N, H), jnp.float32),
    )(h, agg, w1, b1, g1, be1, w2, b2, g2, be2)


@jax.jit
def kernel(x, edge_index, W1, b1, g1, be1, W2, b2, g2, be2):
    src = edge_index[0]
    dst = edge_index[1]
    order = jnp.argsort(dst, stable=True)
    src_s = jnp.concatenate([src[order], jnp.zeros((2 * C,), jnp.int32)])
    dst_s = jnp.concatenate(
        [dst[order], jnp.full((2 * C,), N - 1, jnp.int32)])
    starts = jnp.arange(NW + 1, dtype=jnp.int32) * RPW
    bounds = jnp.searchsorted(dst_s[:E], starts, side="left")
    bounds = jnp.concatenate(
        [bounds.astype(jnp.int32), jnp.zeros((15,), jnp.int32)])

    bpos = jnp.array(_SHARD_BOUNDS, dtype=jnp.int32)
    bdst = dst_s[bpos]
    rstart = jnp.searchsorted(dst_s[:E], bdst, side="left").astype(jnp.int32)
    rend = jnp.searchsorted(dst_s[:E], bdst, side="right").astype(jnp.int32)
    active = (rstart < bpos) & (bpos < rend)
    w_ids = bdst // RPW
    same_prev = jnp.concatenate(
        [jnp.zeros((1,), bool), w_ids[1:] == w_ids[:-1]])
    slot = same_prev.astype(jnp.int32)
    wi = jnp.where(active, w_ids, NW)
    dtab = jnp.zeros((NW + 1, 2, 2), jnp.int32)
    dtab = dtab.at[wi, slot, 0].set(jnp.where(active, bpos, 0))
    dtab = dtab.at[wi, slot, 1].set(jnp.where(active, rend, 0))
    defer = jnp.concatenate(
        [dtab[:NW].reshape(4 * NW), jnp.zeros((16,), jnp.int32)])

    h = x
    for l in range(W1.shape[0]):
        agg = _sc_agg(h, src_s, dst_s, bounds, defer)[:N]
        h = _tc_mlp(h, agg,
                    W1[l], b1[l].reshape(1, H), g1[l].reshape(1, H),
                    be1[l].reshape(1, H),
                    W2[l], b2[l].reshape(1, H), g2[l].reshape(1, H),
                    be2[l].reshape(1, H))
    return h
